# baseline (speedup 1.0000x reference)
import functools

import jax
import jax.numpy as jnp
from jax import lax
from jax.experimental import pallas as pl
from jax.experimental.pallas import tpu as pltpu
from jax.experimental.pallas import tpu_sc as plsc

NUM_CLASSES = 8192
B, K, L, D = 16, 8, 2048, 128
N = B * K * L

NC, NS, LANES = 2, 16, 16
NW = NC * NS
CH = 128
CG = 32
NT = N // CH
NCOMBO = 4
NQ = 64
VPR = D // LANES


def _body(idx_hbm, content_hbm, cb_hbm, fr_hbm, out_hbm,
          idx_v, rows_v, fr_v, cb_v, tab_s,
          sem_idx, sem_g, sem_fr, sem_o):
    sid = lax.axis_index("s")
    wid = sid * NC + lax.axis_index("c")
    k = wid // 4
    ch_base = (wid % 4) * NCOMBO

    @pl.when(sid == 0)
    def _():
        pltpu.sync_copy(content_hbm, tab_s)
    plsc.subcore_barrier()

    def chunk_id(q, e):
        b = (q % 4) * 4 + e
        return (b * K + k) * (L // CH) + ch_base + q // 16

    def start_idx(q, e, slot):
        return pltpu.async_copy(idx_hbm.at[chunk_id(q, e)], idx_v.at[slot],
                                sem_idx.at[slot])

    def start_gather(q, e, slot):
        quarter = (q // 4) % 4
        src = tab_s.at[idx_v.at[slot, 0, pl.ds(quarter * CG, CG)]]
        return pltpu.async_copy(src, rows_v.at[slot], sem_g.at[slot])

    def start_out(q, e, slot):
        quarter = (q // 4) % 4
        nbase = pl.multiple_of(chunk_id(q, e) * CH + quarter * CG, CG)
        return pltpu.async_copy(rows_v.at[slot], out_hbm.at[pl.ds(nbase, CG)],
                                sem_o.at[slot])

    def start_fr(combo):
        l0 = (ch_base + combo) * CH
        return pltpu.async_copy(fr_hbm.at[pl.ds(pl.multiple_of(l0, CH), CH)],
                                fr_v, sem_fr)

    def wait_idx(slot):
        pltpu.make_async_copy(idx_hbm.at[0], idx_v.at[slot],
                              sem_idx.at[slot]).wait()

    def wait_gather(slot):
        src = tab_s.at[idx_v.at[slot, 0, pl.ds(0, CG)]]
        pltpu.make_async_copy(src, rows_v.at[slot], sem_g.at[slot]).wait()

    def wait_out(slot):
        pltpu.make_async_copy(rows_v.at[slot], out_hbm.at[pl.ds(0, CG)],
                              sem_o.at[slot]).wait()

    def wait_fr():
        pltpu.make_async_copy(fr_hbm.at[pl.ds(0, CH)], fr_v, sem_fr).wait()

    pltpu.sync_copy(cb_hbm.at[k], cb_v)
    cbv = [cb_v[0, pl.ds(c * LANES, LANES)] for c in range(VPR)]
    start_fr(0)
    for e in range(4):
        start_idx(0, e, e)
        start_idx(1, e, 4 + e)
    for e in range(4):
        wait_idx(e)
        start_gather(0, e, e)

    for combo in range(NCOMBO):
        wait_fr()
        for quarter in range(4):

            @pl.loop(0, 2)
            def _outer(g2):
                for rq in range(2):
                    q = (combo * 4 + quarter) * 4 + g2 * 2 + rq
                    bank = 4 * rq
                    obank = 4 - bank

                    for e in range(4):
                        wait_gather(bank + e)

                    @pl.when(q + 2 < NQ)
                    def _():
                        for e in range(4):
                            start_idx(q + 2, e, bank + e)

                    @pl.loop(0, CG)
                    def _addrow(row):
                        f = [fr_v[quarter * CG + row, pl.ds(c * LANES, LANES)]
                             for c in range(VPR)]
                        v = [f[c] + cbv[c] for c in range(VPR)]
                        for e in range(4):
                            for c in range(VPR):
                                plsc.addupdate(
                                    rows_v.at[bank + e, row,
                                              pl.ds(c * LANES, LANES)], v[c])

                    @pl.when(q + 1 < NQ)
                    def _():
                        @pl.when(q >= 1)
                        def _():
                            for e in range(4):
                                wait_out(obank + e)
                        for e in range(4):
                            wait_idx(obank + e)
                            start_gather(q + 1, e, obank + e)

                    for e in range(4):
                        start_out(q, e, bank + e)

        if combo + 1 < NCOMBO:
            start_fr(combo + 1)

    for slot in range(8):
        wait_out(slot)


@jax.jit
def _run(idx3d, content_emb, cb3d, frame_emb):
    mesh = plsc.VectorSubcoreMesh(core_axis_name="c", subcore_axis_name="s")
    fn = pl.kernel(
        _body,
        out_type=jax.ShapeDtypeStruct((N, D), jnp.float32),
        mesh=mesh,
        scratch_types=[
            pltpu.VMEM((8, 1, CH), jnp.int32),
            pltpu.VMEM((8, CG, D), jnp.float32),
            pltpu.VMEM((CH, D), jnp.float32),
            pltpu.VMEM((1, D), jnp.float32),
            pltpu.VMEM_SHARED((NUM_CLASSES, D), jnp.float32),
            pltpu.SemaphoreType.DMA((8,)),
            pltpu.SemaphoreType.DMA((8,)),
            pltpu.SemaphoreType.DMA,
            pltpu.SemaphoreType.DMA((8,)),
        ],
    )
    return fn(idx3d, content_emb, cb3d, frame_emb)


def kernel(index, content_emb, codebook_emb, frame_emb):
    idx3d = index.reshape(NT, 1, CH)
    cb3d = codebook_emb.reshape(K, 1, D)
    out = _run(idx3d, content_emb, cb3d, frame_emb)
    return out.reshape(B, K, L, D)


# speedup vs baseline: 1.1797x; 1.1797x over previous
import functools

import jax
import jax.numpy as jnp
from jax import lax
from jax.experimental import pallas as pl
from jax.experimental.pallas import tpu as pltpu
from jax.experimental.pallas import tpu_sc as plsc

NUM_CLASSES = 8192
B, K, L, D = 16, 8, 2048, 128
N = B * K * L

NC, NS, LANES = 2, 16, 16
NW = NC * NS
CH = 128
CG = 64
NT = N // CH
TPW = 2 * (NT // NW)
NCOMBO = 4
VPR = D // LANES


def _body(idx_hbm, content_hbm, cb_hbm, fr_hbm, out_hbm,
          idx_v, rows_v, fr_v, cb_v, tab_s,
          sem_idx, sem_g, sem_fr, sem_o):
    sid = lax.axis_index("s")
    wid = sid * NC + lax.axis_index("c")
    k = wid // 4
    ch_base = (wid % 4) * NCOMBO

    @pl.when(sid == 0)
    def _():
        pltpu.sync_copy(content_hbm, tab_s)
    plsc.subcore_barrier()

    def chunk_id(c):
        return ((c % 16) * K + k) * (L // CH) + ch_base + c // 16

    def sub_base(s):
        return chunk_id(s // 2) * CH + (s % 2) * CG

    def start_idx(c):
        return pltpu.async_copy(idx_hbm.at[chunk_id(c)], idx_v.at[c % 3],
                                sem_idx.at[c % 3])

    def start_gather(s, slot):
        src = tab_s.at[idx_v.at[(s // 2) % 3, 0, pl.ds((s % 2) * CG, CG)]]
        return pltpu.async_copy(src, rows_v.at[slot], sem_g.at[slot])

    def start_out(s, slot):
        nbase = pl.multiple_of(sub_base(s), CG)
        return pltpu.async_copy(rows_v.at[slot], out_hbm.at[pl.ds(nbase, CG)],
                                sem_o.at[slot])

    def start_fr(combo):
        l0 = (ch_base + combo) * CH
        return pltpu.async_copy(fr_hbm.at[pl.ds(pl.multiple_of(l0, CH), CH)],
                                fr_v, sem_fr)

    def wait_idx(c):
        pltpu.make_async_copy(idx_hbm.at[0], idx_v.at[c % 3],
                              sem_idx.at[c % 3]).wait()

    def wait_gather(s, slot):
        src = tab_s.at[idx_v.at[(s // 2) % 3, 0, pl.ds((s % 2) * CG, CG)]]
        pltpu.make_async_copy(src, rows_v.at[slot], sem_g.at[slot]).wait()

    def wait_out(slot):
        pltpu.make_async_copy(rows_v.at[slot], out_hbm.at[pl.ds(0, CG)],
                              sem_o.at[slot]).wait()

    def wait_fr():
        pltpu.make_async_copy(fr_hbm.at[pl.ds(0, CH)], fr_v, sem_fr).wait()

    pltpu.sync_copy(cb_hbm.at[k], cb_v)
    cbv = [cb_v[0, pl.ds(c * LANES, LANES)] for c in range(VPR)]
    start_fr(0)
    start_idx(0)
    start_idx(1)
    wait_idx(0)
    start_gather(0, 0)
    start_gather(1, 1)

    for combo in range(NCOMBO):
        wait_fr()

        @pl.loop(0, 8)
        def _outer(g8):
            for r in range(4):
                s = combo * 32 + g8 * 4 + r
                half = r % 2

                @pl.when(s + 2 < TPW)
                def _():
                    @pl.when(s >= 2)
                    def _():
                        wait_out((r + 2) % 4)
                    if half == 0:
                        wait_idx(s // 2 + 1)
                    start_gather(s + 2, (r + 2) % 4)

                if half == 0:
                    @pl.when(s + 4 < TPW)
                    def _():
                        start_idx(s // 2 + 2)

                wait_gather(s, r)

                @pl.loop(0, CG, unroll=2)
                def _addrow(row):
                    f = [fr_v[half * CG + row, pl.ds(c * LANES, LANES)]
                         for c in range(VPR)]
                    v = [f[c] + cbv[c] for c in range(VPR)]
                    for c in range(VPR):
                        plsc.addupdate(
                            rows_v.at[r, row, pl.ds(c * LANES, LANES)], v[c])

                start_out(s, r)

        if combo + 1 < NCOMBO:
            start_fr(combo + 1)

    for r in range(4):
        wait_out(r)


@jax.jit
def _run(idx3d, content_emb, cb3d, frame_emb):
    mesh = plsc.VectorSubcoreMesh(core_axis_name="c", subcore_axis_name="s")
    fn = pl.kernel(
        _body,
        out_type=jax.ShapeDtypeStruct((N, D), jnp.float32),
        mesh=mesh,
        scratch_types=[
            pltpu.VMEM((3, 1, CH), jnp.int32),
            pltpu.VMEM((4, CG, D), jnp.float32),
            pltpu.VMEM((CH, D), jnp.float32),
            pltpu.VMEM((1, D), jnp.float32),
            pltpu.VMEM_SHARED((NUM_CLASSES, D), jnp.float32),
            pltpu.SemaphoreType.DMA((3,)),
            pltpu.SemaphoreType.DMA((4,)),
            pltpu.SemaphoreType.DMA,
            pltpu.SemaphoreType.DMA((4,)),
        ],
    )
    return fn(idx3d, content_emb, cb3d, frame_emb)


def kernel(index, content_emb, codebook_emb, frame_emb):
    idx3d = index.reshape(NT, 1, CH)
    cb3d = codebook_emb.reshape(K, 1, D)
    out = _run(idx3d, content_emb, cb3d, frame_emb)
    return out.reshape(B, K, L, D)
